# baseline (speedup 1.0000x reference)
import functools

import jax
import jax.numpy as jnp
from jax import lax
from jax.experimental import pallas as pl
from jax.experimental.pallas import tpu as pltpu
from jax.experimental.pallas import tpu_sc as plsc

VOCAB_N = 1000000
D_MODEL = 64
T_LEN = 200
B_LEN = 4096
NUM_TILES = 32
B_BLK = B_LEN // NUM_TILES
TOK_PER_TILE = T_LEN * B_BLK

_mesh = plsc.VectorSubcoreMesh(core_axis_name="c", subcore_axis_name="s")

@functools.partial(
    pl.kernel,
    mesh=_mesh,
    out_type=jax.ShapeDtypeStruct((T_LEN, D_MODEL, B_LEN), jnp.float32),
    compiler_params=pltpu.CompilerParams(needs_layout_passes=False,
                                         use_tc_tiling_on_sc=False),
    scratch_types=[
        pltpu.VMEM((TOK_PER_TILE,), jnp.int32),
        pltpu.VMEM((TOK_PER_TILE,), jnp.int32),
        pltpu.VMEM((B_BLK, 128), jnp.float32),
        pltpu.VMEM((B_BLK, 128), jnp.float32),
        pltpu.VMEM((D_MODEL, B_BLK), jnp.float32),
        pltpu.VMEM((D_MODEL, B_BLK), jnp.float32),
        pltpu.SemaphoreType.DMA,
        pltpu.SemaphoreType.DMA,
        pltpu.SemaphoreType.DMA,
        pltpu.SemaphoreType.DMA,
    ],
)
def _pair_gather_kernel(xpair_hbm, xoff_hbm, table2_hbm, out_hbm,
                        idx1, off1, gbufa, gbufb, tbufa, tbufb,
                        gsema, gsemb, wsema, wsemb):
    cid = lax.axis_index("c")
    sid = lax.axis_index("s")
    wid = cid * 16 + sid
    tok0 = pl.multiple_of(wid * TOK_PER_TILE, 128)
    b0 = pl.multiple_of(wid * B_BLK, 128)
    gbufs = (gbufa, gbufb)
    tbufs = (tbufa, tbufb)
    gsems = (gsema, gsemb)
    wsems = (wsema, wsemb)

    pltpu.sync_copy(xpair_hbm.at[pl.ds(tok0, TOK_PER_TILE)], idx1)
    pltpu.sync_copy(xoff_hbm.at[pl.ds(tok0, TOK_PER_TILE)], off1)

    def idx_slice(t):
        return idx1.at[pl.ds(pl.multiple_of(t * B_BLK, 128), B_BLK)]

    def gather_copy(t, p):
        return pltpu.make_async_copy(table2_hbm.at[idx_slice(t)],
                                     gbufs[p], gsems[p])

    def write_copy(t, p):
        return pltpu.make_async_copy(
            tbufs[p], out_hbm.at[t, :, pl.ds(b0, B_BLK)], wsems[p])

    lanes16 = lax.iota(jnp.int32, 16)

    def extract(t, p):
        gbuf = gbufs[p]
        tbuf = tbufs[p]
        base = t * B_BLK
        rows = [lanes16 + (16 * k) for k in range(8)]
        xos = [off1[pl.ds(base + 16 * k, 16)] for k in range(8)]
        for d in range(D_MODEL):
            cols = [xos[k] + d for k in range(8)]
            vals = [plsc.load_gather(gbuf, [rows[k], cols[k]])
                    for k in range(8)]
            for k in range(8):
                tbuf[d, pl.ds(16 * k, 16)] = vals[k]

    gather_copy(0, 0).start()
    gather_copy(1, 1).start()

    def body(i, carry):
        t0 = 2 * i
        for q in range(2):
            tt = t0 + q
            gather_copy(tt, q).wait()

            @pl.when(tt >= 2)
            def _(tt=tt, q=q):
                write_copy(tt - 2, q).wait()

            extract(tt, q)
            write_copy(tt, q).start()

            @pl.when(tt + 2 < T_LEN)
            def _(tt=tt, q=q):
                gather_copy(tt + 2, q).start()
        return carry

    lax.fori_loop(0, T_LEN // 2, body, 0)

    for tt in (T_LEN - 2, T_LEN - 1):
        write_copy(tt, tt % 2).wait()


def kernel(x, table):
    table2 = table.reshape(VOCAB_N // 2, 2 * D_MODEL)
    xr = x.T.reshape(T_LEN, NUM_TILES, B_BLK).transpose(1, 0, 2).reshape(-1)
    xr = xr.astype(jnp.int32)
    xpair = xr >> 1
    xoff = (xr & 1) * D_MODEL
    out3 = _pair_gather_kernel(xpair, xoff, table2)
    return out3.transpose(2, 0, 1)


# speedup vs baseline: 1.0113x; 1.0113x over previous
import functools

import jax
import jax.numpy as jnp
from jax import lax
from jax.experimental import pallas as pl
from jax.experimental.pallas import tpu as pltpu
from jax.experimental.pallas import tpu_sc as plsc

VOCAB_N = 1000000
D_MODEL = 64
T_LEN = 200
B_LEN = 4096
NUM_TILES = 32
B_BLK = B_LEN // NUM_TILES
TOK_PER_TILE = T_LEN * B_BLK
NRING = 4

_mesh = plsc.VectorSubcoreMesh(core_axis_name="c", subcore_axis_name="s")


@functools.partial(
    pl.kernel,
    mesh=_mesh,
    out_type=jax.ShapeDtypeStruct((T_LEN, D_MODEL, B_LEN), jnp.float32),
    compiler_params=pltpu.CompilerParams(needs_layout_passes=False,
                                         use_tc_tiling_on_sc=False),
    scratch_types=[pltpu.VMEM((TOK_PER_TILE,), jnp.int32)]
    + [pltpu.VMEM((B_BLK, D_MODEL), jnp.float32)] * NRING
    + [pltpu.VMEM((D_MODEL, B_BLK), jnp.float32)] * 2
    + [pltpu.SemaphoreType.DMA] * (NRING + 2),
)
def _gather_kernel(x_hbm, table_hbm, out_hbm, idx1, *bufs):
    gbufs = bufs[:NRING]
    tbufs = bufs[NRING:NRING + 2]
    gsems = bufs[NRING + 2:2 * NRING + 2]
    wsems = bufs[2 * NRING + 2:]
    cid = lax.axis_index("c")
    sid = lax.axis_index("s")
    wid = cid * 16 + sid
    tok0 = pl.multiple_of(wid * TOK_PER_TILE, 128)
    b0 = pl.multiple_of(wid * B_BLK, 128)

    pltpu.sync_copy(x_hbm.at[pl.ds(tok0, TOK_PER_TILE)], idx1)

    def gather_copy(t, g):
        sl = idx1.at[pl.ds(pl.multiple_of(t * B_BLK, 128), B_BLK)]
        return pltpu.make_async_copy(table_hbm.at[sl], gbufs[g], gsems[g])

    def write_copy(t, p):
        return pltpu.make_async_copy(
            tbufs[p], out_hbm.at[t, :, pl.ds(b0, B_BLK)], wsems[p])

    lanes16 = lax.iota(jnp.int32, 16)
    rows = [lanes16 + (16 * k) for k in range(8)]
    zero16 = lanes16 * 0

    def extract(g, p):
        gbuf = gbufs[g]
        tbuf = tbufs[p]
        for d in range(D_MODEL):
            cols = zero16 + d
            vals = [plsc.load_gather(gbuf, [rows[k], cols]) for k in range(8)]
            for k in range(8):
                tbuf[d, pl.ds(16 * k, 16)] = vals[k]

    for t in range(NRING):
        gather_copy(t, t).start()

    def body(i, carry):
        t0 = NRING * i
        for q in range(NRING):
            tt = t0 + q
            p = q % 2
            gather_copy(tt, q).wait()

            @pl.when(tt >= 2)
            def _(tt=tt, p=p):
                write_copy(tt - 2, p).wait()

            extract(q, p)
            write_copy(tt, p).start()

            @pl.when(tt + NRING < T_LEN)
            def _(tt=tt, q=q):
                gather_copy(tt + NRING, q).start()
        return carry

    lax.fori_loop(0, T_LEN // NRING, body, 0)

    for tt in (T_LEN - 2, T_LEN - 1):
        write_copy(tt, tt % 2).wait()


def kernel(x, table):
    xr = x.T.reshape(T_LEN, NUM_TILES, B_BLK).transpose(1, 0, 2).reshape(-1)
    out3 = _gather_kernel(xr.astype(jnp.int32), table)
    return out3.transpose(2, 0, 1)


# speedup vs baseline: 1.3530x; 1.3379x over previous
import functools

import jax
import jax.numpy as jnp
from jax import lax
from jax.experimental import pallas as pl
from jax.experimental.pallas import tpu as pltpu
from jax.experimental.pallas import tpu_sc as plsc

VOCAB_N = 1000000
D_MODEL = 64
T_LEN = 200
B_LEN = 4096
NUM_TILES = 32
B_BLK = B_LEN // NUM_TILES
TOK_PER_TILE = T_LEN * B_BLK
NRING = 2

_mesh = plsc.VectorSubcoreMesh(core_axis_name="c", subcore_axis_name="s")


@functools.partial(
    pl.kernel,
    mesh=_mesh,
    out_type=jax.ShapeDtypeStruct((T_LEN, D_MODEL, B_LEN), jnp.float32),
    compiler_params=pltpu.CompilerParams(needs_layout_passes=False,
                                         use_tc_tiling_on_sc=False),
    scratch_types=[pltpu.VMEM((TOK_PER_TILE,), jnp.int32)]
    + [pltpu.VMEM((B_BLK, D_MODEL), jnp.float32)] * NRING
    + [pltpu.VMEM((D_MODEL, B_BLK + 1), jnp.float32)] * 2
    + [pltpu.SemaphoreType.DMA] * (NRING + 2),
)
def _gather_kernel(x_hbm, table_hbm, out_hbm, idx1, *bufs):
    gbufs = bufs[:NRING]
    tbufs = bufs[NRING:NRING + 2]
    gsems = bufs[NRING + 2:2 * NRING + 2]
    wsems = bufs[2 * NRING + 2:]
    cid = lax.axis_index("c")
    sid = lax.axis_index("s")
    wid = cid * 16 + sid
    tok0 = pl.multiple_of(wid * TOK_PER_TILE, 128)
    b0 = pl.multiple_of(wid * B_BLK, 128)

    pltpu.sync_copy(x_hbm.at[pl.ds(tok0, TOK_PER_TILE)], idx1)

    def gather_copy(t, g):
        sl = idx1.at[pl.ds(pl.multiple_of(t * B_BLK, 128), B_BLK)]
        return pltpu.make_async_copy(table_hbm.at[sl], gbufs[g], gsems[g])

    def write_copy(t, p):
        return pltpu.make_async_copy(
            tbufs[p].at[:, pl.ds(0, B_BLK)],
            out_hbm.at[t, :, pl.ds(b0, B_BLK)], wsems[p])

    lanes16 = lax.iota(jnp.int32, 16)
    drows = [lanes16 + (16 * j) for j in range(4)]
    zero16 = lanes16 * 0

    def extract(g, p):
        gbuf = gbufs[g]
        tbuf = tbufs[p]
        for tok in range(B_BLK):
            cols = zero16 + tok
            for j in range(4):
                val = gbuf[tok, pl.ds(16 * j, 16)]
                plsc.store_scatter(tbuf, [drows[j], cols], val)

    for t in range(NRING):
        gather_copy(t, t).start()

    def body(i, carry):
        t0 = NRING * i
        for q in range(NRING):
            tt = t0 + q
            p = q % 2
            gather_copy(tt, q).wait()

            @pl.when(tt >= 2)
            def _(tt=tt, p=p):
                write_copy(tt - 2, p).wait()

            extract(q, p)
            write_copy(tt, p).start()

            @pl.when(tt + NRING < T_LEN)
            def _(tt=tt, q=q):
                gather_copy(tt + NRING, q).start()
        return carry

    lax.fori_loop(0, T_LEN // NRING, body, 0)

    for tt in (T_LEN - 2, T_LEN - 1):
        write_copy(tt, tt % 2).wait()


def kernel(x, table):
    xr = x.T.reshape(T_LEN, NUM_TILES, B_BLK).transpose(1, 0, 2).reshape(-1)
    out3 = _gather_kernel(xr.astype(jnp.int32), table)
    return out3.transpose(2, 0, 1)
